# baseline (speedup 1.0000x reference)
import functools

import jax
import jax.numpy as jnp
from jax import lax
from jax.experimental import pallas as pl
from jax.experimental.pallas import tpu as pltpu
from jax.experimental.pallas import tpu_sc as plsc

N = 10000
E = 320000
D = 128
K = 8

NC = 2
NS = 16
HALF = D // NC
EPW = E // NS
C = 80
NCHUNK = EPW // C
ROFF = 624
RSZ = 640
ZR = 80
NB = 5
LOOK = 3


_BCAST_DN = lax.GatherDimensionNumbers(
    offset_dims=(), collapsed_slice_dims=(0,), start_index_map=(0,))


def _sc_prop(xs, src_r, dst_r, w_r):
    mesh = plsc.VectorSubcoreMesh(core_axis_name="c", subcore_axis_name="s")

    @functools.partial(
        pl.kernel,
        out_type=jax.ShapeDtypeStruct((NC * N, HALF), jnp.float32),
        mesh=mesh,
        compiler_params=pltpu.CompilerParams(use_tc_tiling_on_sc=False),
        scratch_types=[
            pltpu.VMEM((NCHUNK, C), jnp.int32),
            pltpu.VMEM((NCHUNK, C), jnp.int32),
            pltpu.VMEM((NCHUNK, C), jnp.float32),
            [pltpu.VMEM((C, HALF), jnp.float32) for _ in range(NB)],
            pltpu.VMEM((ZR, HALF), jnp.float32),
            pltpu.VMEM_SHARED((N, HALF), jnp.float32),
            [pltpu.SemaphoreType.DMA for _ in range(NB)],
            [pltpu.SemaphoreType.DMA for _ in range(NB)],
        ],
    )
    def k(xs_hbm, src_hbm, dst_hbm, w_hbm, out_hbm,
          src_v, dst_v, w_v, rbufs, zbuf, acc_sh, gsems, ssems):
        c = lax.axis_index("c")
        s = lax.axis_index("s")
        r0 = s * ROFF
        ro = c * N + r0

        pltpu.sync_copy(src_hbm.at[s], src_v)
        pltpu.sync_copy(dst_hbm.at[s], dst_v)
        pltpu.sync_copy(w_hbm.at[s], w_v)

        cN = jnp.full((16,), c * N, jnp.int32)

        def adj(kk, _):
            for g in range(C // 16):
                sl = pl.ds(g * 16, 16)
                src_v[kk, sl] = src_v[kk, sl] + cN
            return 0

        lax.fori_loop(0, NCHUNK, adj, 0)

        for q in range(RSZ // ZR):
            pltpu.sync_copy(xs_hbm.at[pl.ds(ro + q * ZR, ZR)], zbuf)
            pltpu.sync_copy(zbuf, out_hbm.at[pl.ds(ro + q * ZR, ZR)])

        z16 = jnp.zeros((16,), jnp.float32)

        def zero(kk, _):
            for g in range(HALF // 16):
                zbuf[kk, pl.ds(g * 16, 16)] = z16
            return 0

        lax.fori_loop(0, ZR, zero, 0)
        plsc.subcore_barrier()

        def scale(buf, ck):
            for gg in range(C // 16):
                wv = w_v[ck, pl.ds(gg * 16, 16)]
                for el in range(16):
                    e = gg * 16 + el
                    wbc = lax.gather(
                        wv, jnp.full((16, 1), el, jnp.int32),
                        _BCAST_DN, slice_sizes=(1,),
                        mode=lax.GatherScatterMode.PROMISE_IN_BOUNDS)
                    for g in range(HALF // 16):
                        sl = pl.ds(g * 16, 16)
                        buf[e, sl] = buf[e, sl] * wbc

        def round_body(t, _):
            for q in range(RSZ // ZR):
                pltpu.sync_copy(zbuf, acc_sh.at[pl.ds(r0 + q * ZR, ZR)])
            plsc.subcore_barrier()

            for b in range(LOOK):
                pltpu.async_copy(out_hbm.at[src_v.at[b]], rbufs[b], gsems[b])

            def chunk_group(kk, __):
                for b in range(NB):
                    ck = kk * NB + b
                    pltpu.make_async_copy(
                        out_hbm.at[src_v.at[ck]], rbufs[b], gsems[b]).wait()
                    scale(rbufs[b], ck)
                    pltpu.async_copy(
                        rbufs[b], acc_sh.at[dst_v.at[ck]], ssems[b], add=True)
                    nb = (b + LOOK) % NB
                    nck = ck + LOOK

                    @pl.when(nck < NCHUNK)
                    def _():
                        @pl.when(ck >= NB - LOOK)
                        def _():
                            pltpu.make_async_copy(
                                rbufs[nb], acc_sh.at[dst_v.at[nck - NB]],
                                ssems[nb]).wait()
                        pltpu.async_copy(
                            out_hbm.at[src_v.at[nck]], rbufs[nb], gsems[nb])
                return 0

            lax.fori_loop(0, NCHUNK // NB, chunk_group, 0)
            for b in range(NB):
                pltpu.make_async_copy(
                    rbufs[b], acc_sh.at[dst_v.at[NCHUNK - NB + b]],
                    ssems[b]).wait()
            plsc.subcore_barrier()
            pltpu.sync_copy(acc_sh.at[pl.ds(r0, RSZ)], out_hbm.at[pl.ds(ro, RSZ)])
            plsc.subcore_barrier()
            return 0

        lax.fori_loop(0, K, round_body, 0)

    return k(xs, src_r, dst_r, w_r)


def kernel(x, edge_index, edge_weight):
    xs = x.reshape(N, NC, HALF).transpose(1, 0, 2).reshape(NC * N, HALF)
    dst0 = edge_index[0].astype(jnp.int32)
    src0 = edge_index[1].astype(jnp.int32)
    order = jnp.argsort(src0)
    dst = dst0[order].reshape(NS, NCHUNK, C)
    src = src0[order].reshape(NS, NCHUNK, C)
    w = edge_weight[order].reshape(NS, NCHUNK, C)
    out = _sc_prop(xs, src, dst, w)
    return out.reshape(NC, N, HALF).transpose(1, 0, 2).reshape(N, D)


# speedup vs baseline: 2.3073x; 2.3073x over previous
import functools

import jax
import jax.numpy as jnp
from jax import lax
from jax.experimental import pallas as pl
from jax.experimental.pallas import tpu as pltpu
from jax.experimental.pallas import tpu_sc as plsc

N = 10000
E = 320000
D = 128
K = 8

NC = 2
NS = 16
HALF = D // NC
EPW = E // NS
C = 80
NCHUNK = EPW // C
ROFF = 624
RSZ = 640
ZR = 80
NB = 5
LOOK = 4


_BCAST_DN = lax.GatherDimensionNumbers(
    offset_dims=(), collapsed_slice_dims=(0,), start_index_map=(0,))


def _sc_prop(xs, src_r, dst_r, w_r):
    mesh = plsc.VectorSubcoreMesh(core_axis_name="c", subcore_axis_name="s")

    @functools.partial(
        pl.kernel,
        out_type=jax.ShapeDtypeStruct((NC * N, HALF), jnp.float32),
        mesh=mesh,
        compiler_params=pltpu.CompilerParams(use_tc_tiling_on_sc=False),
        scratch_types=[
            pltpu.VMEM((NCHUNK, C), jnp.int32),
            pltpu.VMEM((NCHUNK, C), jnp.int32),
            pltpu.VMEM((NCHUNK, C), jnp.float32),
            [pltpu.VMEM((C, HALF), jnp.float32) for _ in range(NB)],
            pltpu.VMEM((ZR, HALF), jnp.float32),
            pltpu.VMEM_SHARED((N, HALF), jnp.float32),
            [pltpu.SemaphoreType.DMA for _ in range(NB)],
            [pltpu.SemaphoreType.DMA for _ in range(NB)],
        ],
    )
    def k(xs_hbm, src_hbm, dst_hbm, w_hbm, out_hbm,
          src_v, dst_v, w_v, rbufs, zbuf, acc_sh, gsems, ssems):
        c = lax.axis_index("c")
        s = lax.axis_index("s")
        r0 = s * ROFF
        ro = c * N + r0

        pltpu.sync_copy(src_hbm.at[s], src_v)
        pltpu.sync_copy(dst_hbm.at[s], dst_v)
        pltpu.sync_copy(w_hbm.at[s], w_v)

        cN = jnp.full((16,), c * N, jnp.int32)

        def adj(kk, _):
            for g in range(C // 16):
                sl = pl.ds(g * 16, 16)
                src_v[kk, sl] = src_v[kk, sl] + cN
            return 0

        lax.fori_loop(0, NCHUNK, adj, 0)

        for q in range(RSZ // ZR):
            pltpu.sync_copy(xs_hbm.at[pl.ds(ro + q * ZR, ZR)], zbuf)
            pltpu.sync_copy(zbuf, out_hbm.at[pl.ds(ro + q * ZR, ZR)])

        z16 = jnp.zeros((16,), jnp.float32)

        def zero(kk, _):
            for g in range(HALF // 16):
                zbuf[kk, pl.ds(g * 16, 16)] = z16
            return 0

        lax.fori_loop(0, ZR, zero, 0)
        plsc.subcore_barrier()

        def scale(buf, ck):
            for gg in range(C // 16):
                wv = w_v[ck, pl.ds(gg * 16, 16)]
                for el in range(16):
                    e = gg * 16 + el
                    wbc = lax.gather(
                        wv, jnp.full((16, 1), el, jnp.int32),
                        _BCAST_DN, slice_sizes=(1,),
                        mode=lax.GatherScatterMode.PROMISE_IN_BOUNDS)
                    for g in range(HALF // 16):
                        sl = pl.ds(g * 16, 16)
                        buf[e, sl] = buf[e, sl] * wbc

        def round_body(t, _):
            for q in range(RSZ // ZR):
                pltpu.sync_copy(zbuf, acc_sh.at[pl.ds(r0 + q * ZR, ZR)])
            plsc.subcore_barrier()

            for b in range(LOOK):
                pltpu.async_copy(out_hbm.at[src_v.at[b]], rbufs[b], gsems[b])

            def chunk_group(kk, __):
                for b in range(NB):
                    ck = kk * NB + b
                    pltpu.make_async_copy(
                        out_hbm.at[src_v.at[ck]], rbufs[b], gsems[b]).wait()
                    scale(rbufs[b], ck)
                    pltpu.async_copy(
                        rbufs[b], acc_sh.at[dst_v.at[ck]], ssems[b], add=True)
                    nb = (b + LOOK) % NB
                    nck = ck + LOOK

                    @pl.when(nck < NCHUNK)
                    def _():
                        @pl.when(ck >= NB - LOOK)
                        def _():
                            pltpu.make_async_copy(
                                rbufs[nb], acc_sh.at[dst_v.at[nck - NB]],
                                ssems[nb]).wait()
                        pltpu.async_copy(
                            out_hbm.at[src_v.at[nck]], rbufs[nb], gsems[nb])
                return 0

            lax.fori_loop(0, NCHUNK // NB, chunk_group, 0)
            for b in range(NB):
                pltpu.make_async_copy(
                    rbufs[b], acc_sh.at[dst_v.at[NCHUNK - NB + b]],
                    ssems[b]).wait()
            plsc.subcore_barrier()
            pltpu.sync_copy(acc_sh.at[pl.ds(r0, RSZ)], out_hbm.at[pl.ds(ro, RSZ)])
            plsc.subcore_barrier()
            return 0

        lax.fori_loop(0, K, round_body, 0)

    return k(xs, src_r, dst_r, w_r)


def kernel(x, edge_index, edge_weight):
    xs = x.reshape(N, NC, HALF).transpose(1, 0, 2).reshape(NC * N, HALF)
    dst = edge_index[0].astype(jnp.int32).reshape(NS, NCHUNK, C)
    src = edge_index[1].astype(jnp.int32).reshape(NS, NCHUNK, C)
    w = edge_weight.reshape(NS, NCHUNK, C)
    out = _sc_prop(xs, src, dst, w)
    return out.reshape(NC, N, HALF).transpose(1, 0, 2).reshape(N, D)


# speedup vs baseline: 2.3500x; 1.0185x over previous
import functools

import jax
import jax.numpy as jnp
from jax import lax
from jax.experimental import pallas as pl
from jax.experimental.pallas import tpu as pltpu
from jax.experimental.pallas import tpu_sc as plsc

N = 10000
E = 320000
D = 128
K = 8

NC = 2
NS = 16
HALF = D // NC
EPW = E // NS
C = 80
NCHUNK = EPW // C
ROFF = 624
RSZ = 640
ZR = 80
NB = 5
LOOK = 4


_BCAST_DN = lax.GatherDimensionNumbers(
    offset_dims=(), collapsed_slice_dims=(0,), start_index_map=(0,))


def _sc_prop(xs, src_r, dst_r, w_r):
    mesh = plsc.VectorSubcoreMesh(core_axis_name="c", subcore_axis_name="s")

    @functools.partial(
        pl.kernel,
        out_type=jax.ShapeDtypeStruct((NC * N, HALF), jnp.float32),
        mesh=mesh,
        compiler_params=pltpu.CompilerParams(use_tc_tiling_on_sc=False),
        scratch_types=[
            pltpu.VMEM((NCHUNK, C), jnp.int32),
            pltpu.VMEM((NCHUNK, C), jnp.int32),
            pltpu.VMEM((NCHUNK, C), jnp.float32),
            [pltpu.VMEM((C, HALF), jnp.float32) for _ in range(NB)],
            pltpu.VMEM((ZR, HALF), jnp.float32),
            pltpu.VMEM_SHARED((N, HALF), jnp.float32),
            [pltpu.SemaphoreType.DMA for _ in range(NB)],
            [pltpu.SemaphoreType.DMA for _ in range(NB)],
        ],
    )
    def k(xs_hbm, src_hbm, dst_hbm, w_hbm, out_hbm,
          src_v, dst_v, w_v, rbufs, zbuf, acc_sh, gsems, ssems):
        c = lax.axis_index("c")
        s = lax.axis_index("s")
        r0 = s * ROFF
        ro = c * N + r0

        pltpu.sync_copy(src_hbm.at[s], src_v)
        pltpu.sync_copy(dst_hbm.at[s], dst_v)
        pltpu.sync_copy(w_hbm.at[s], w_v)

        cN = jnp.full((16,), c * N, jnp.int32)

        def adj(kk, _):
            for g in range(C // 16):
                sl = pl.ds(g * 16, 16)
                src_v[kk, sl] = src_v[kk, sl] + cN
            return 0

        lax.fori_loop(0, NCHUNK, adj, 0)

        for q in range(RSZ // ZR):
            pltpu.sync_copy(xs_hbm.at[pl.ds(ro + q * ZR, ZR)], zbuf)
            pltpu.sync_copy(zbuf, out_hbm.at[pl.ds(ro + q * ZR, ZR)])

        z16 = jnp.zeros((16,), jnp.float32)

        def zero(kk, _):
            for g in range(HALF // 16):
                zbuf[kk, pl.ds(g * 16, 16)] = z16
            return 0

        lax.fori_loop(0, ZR, zero, 0)
        plsc.subcore_barrier()

        def scale(buf, ck):
            for gg in range(C // 16):
                wv = w_v[ck, pl.ds(gg * 16, 16)]
                for el in range(16):
                    e = gg * 16 + el
                    wbc = lax.gather(
                        wv, jnp.full((16, 1), el, jnp.int32),
                        _BCAST_DN, slice_sizes=(1,),
                        mode=lax.GatherScatterMode.PROMISE_IN_BOUNDS)
                    for g in range(HALF // 16):
                        sl = pl.ds(g * 16, 16)
                        buf[e, sl] = buf[e, sl] * wbc

        def round_body(t, _):
            for b in range(LOOK):
                pltpu.async_copy(out_hbm.at[src_v.at[b]], rbufs[b], gsems[b])
            for q in range(RSZ // ZR):
                pltpu.sync_copy(zbuf, acc_sh.at[pl.ds(r0 + q * ZR, ZR)])
            plsc.subcore_barrier()

            def chunk_group(kk, __):
                for b in range(NB):
                    ck = kk * NB + b
                    pltpu.make_async_copy(
                        out_hbm.at[src_v.at[ck]], rbufs[b], gsems[b]).wait()
                    scale(rbufs[b], ck)
                    pltpu.async_copy(
                        rbufs[b], acc_sh.at[dst_v.at[ck]], ssems[b], add=True)
                    nb = (b + LOOK) % NB
                    nck = ck + LOOK

                    @pl.when(nck < NCHUNK)
                    def _():
                        @pl.when(ck >= NB - LOOK)
                        def _():
                            pltpu.make_async_copy(
                                rbufs[nb], acc_sh.at[dst_v.at[nck - NB]],
                                ssems[nb]).wait()
                        pltpu.async_copy(
                            out_hbm.at[src_v.at[nck]], rbufs[nb], gsems[nb])
                return 0

            lax.fori_loop(0, NCHUNK // NB, chunk_group, 0)
            for b in range(NB):
                pltpu.make_async_copy(
                    rbufs[b], acc_sh.at[dst_v.at[NCHUNK - NB + b]],
                    ssems[b]).wait()
            plsc.subcore_barrier()
            pltpu.sync_copy(acc_sh.at[pl.ds(r0, RSZ)], out_hbm.at[pl.ds(ro, RSZ)])
            plsc.subcore_barrier()
            return 0

        lax.fori_loop(0, K, round_body, 0)

    return k(xs, src_r, dst_r, w_r)


def kernel(x, edge_index, edge_weight):
    xs = x.reshape(N, NC, HALF).transpose(1, 0, 2).reshape(NC * N, HALF)
    dst = edge_index[0].astype(jnp.int32).reshape(NS, NCHUNK, C)
    src = edge_index[1].astype(jnp.int32).reshape(NS, NCHUNK, C)
    w = edge_weight.reshape(NS, NCHUNK, C)
    out = _sc_prop(xs, src, dst, w)
    return out.reshape(NC, N, HALF).transpose(1, 0, 2).reshape(N, D)
